# baseline (speedup 1.0000x reference)
import functools

import jax
import jax.numpy as jnp
from jax import lax
from jax.experimental import pallas as pl
from jax.experimental.pallas import tpu as pltpu
from jax.experimental.pallas import tpu_sc as plsc

VOCAB = 1000000
MAXLEN = 200
EMBED = 64
BATCH = 4096

N = BATCH * MAXLEN
NC = 2
NS = 16
NW = NC * NS
RW = N // NW
SUB = 100
CHUNK = 400
NSUB = CHUNK // SUB
NCHUNK = RW // CHUNK
XROWS = N // SUB
NREP = CHUNK // MAXLEN


_mesh = plsc.VectorSubcoreMesh(core_axis_name="c", subcore_axis_name="s")


@functools.partial(
    pl.kernel,
    mesh=_mesh,
    out_type=jax.ShapeDtypeStruct((N, EMBED), jnp.float32),
    scratch_types=[
        pltpu.VMEM((2 * NSUB, SUB), jnp.int32),
        pltpu.VMEM((CHUNK, EMBED), jnp.float32),
        pltpu.VMEM((CHUNK, EMBED), jnp.float32),
        pltpu.VMEM((MAXLEN, EMBED), jnp.float32),
        pltpu.SemaphoreType.DMA,
        pltpu.SemaphoreType.DMA,
        pltpu.SemaphoreType.DMA,
        pltpu.SemaphoreType.DMA,
    ],
    compiler_params=pltpu.CompilerParams(use_tc_tiling_on_sc=False),
)
def _embed_kernel(x_hbm, tok_hbm, pos_hbm, out_hbm,
                  idx_v, buf0, buf1, pos_v, gsem0, gsem1, osem0, osem1):
    wid = lax.axis_index("s") * NC + lax.axis_index("c")
    idx_base = wid * (RW // SUB)
    row_base = wid * RW
    bufs = (buf0, buf1)
    gsems = (gsem0, gsem1)
    osems = (osem0, osem1)

    pltpu.sync_copy(pos_hbm, pos_v)

    def issue_gather(c, b):
        pltpu.sync_copy(x_hbm.at[pl.ds(idx_base + c * NSUB, NSUB)],
                        idx_v.at[pl.ds(b * NSUB, NSUB)])
        for j in range(NSUB):
            pltpu.async_copy(
                tok_hbm.at[idx_v.at[b * NSUB + j]],
                bufs[b].at[pl.ds(j * SUB, SUB)],
                gsems[b],
            )

    def wait_gather(b):
        for j in range(NSUB):
            pltpu.make_async_copy(
                tok_hbm.at[idx_v.at[b * NSUB + j]],
                bufs[b].at[pl.ds(j * SUB, SUB)],
                gsems[b],
            ).wait()

    def issue_out(c, b):
        pltpu.async_copy(bufs[b], out_hbm.at[pl.ds(row_base + c * CHUNK, CHUNK)],
                         osems[b])

    def wait_out(b):
        pltpu.make_async_copy(bufs[b], out_hbm.at[pl.ds(row_base, CHUNK)],
                              osems[b]).wait()

    issue_gather(0, 0)

    def pair_body(i, carry):
        c0 = 2 * i
        for b in range(2):
            c = c0 + b
            nb = 1 - b
            nxt = c + 1

            @pl.when(nxt < NCHUNK)
            def _prefetch():
                @pl.when(c >= 1)
                def _reclaim():
                    wait_out(nb)
                issue_gather(nxt, nb)

            wait_gather(b)
            buf = bufs[b]

            def pos_add(p, carry2):
                for q in range(EMBED // 16):
                    sl = pl.ds(q * 16, 16)
                    pv = pos_v[p, sl]
                    for rep in range(NREP):
                        r = rep * MAXLEN + p
                        buf[r, sl] = buf[r, sl] + pv
                return carry2

            lax.fori_loop(0, MAXLEN, pos_add, 0)
            issue_out(c, b)
        return carry

    lax.fori_loop(0, NCHUNK // 2, pair_body, 0)
    wait_out(0)
    wait_out(1)


def kernel(x, token_table, pos_table):
    x_resh = x.reshape(XROWS, SUB).astype(jnp.int32)
    out = _embed_kernel(x_resh, token_table, pos_table)
    return out.reshape(BATCH, MAXLEN, EMBED)


# speedup vs baseline: 1.0023x; 1.0023x over previous
import functools

import jax
import jax.numpy as jnp
from jax import lax
from jax.experimental import pallas as pl
from jax.experimental.pallas import tpu as pltpu
from jax.experimental.pallas import tpu_sc as plsc

VOCAB = 1000000
MAXLEN = 200
EMBED = 64
BATCH = 4096

NC = 2
NS = 16
NW = NC * NS
BW = BATCH // NW
BPC = 2
NCHUNK = BW // BPC
SUBS = ((0, 96), (96, 104))


_mesh = plsc.VectorSubcoreMesh(core_axis_name="c", subcore_axis_name="s")


@functools.partial(
    pl.kernel,
    mesh=_mesh,
    out_type=jax.ShapeDtypeStruct((BATCH, MAXLEN, EMBED), jnp.float32),
    scratch_types=[
        pltpu.VMEM((2 * BPC, MAXLEN), jnp.int32),
        pltpu.VMEM((BPC, MAXLEN, EMBED), jnp.float32),
        pltpu.VMEM((BPC, MAXLEN, EMBED), jnp.float32),
        pltpu.VMEM((MAXLEN, EMBED), jnp.float32),
        pltpu.SemaphoreType.DMA,
        pltpu.SemaphoreType.DMA,
        pltpu.SemaphoreType.DMA,
        pltpu.SemaphoreType.DMA,
    ],
    compiler_params=pltpu.CompilerParams(use_tc_tiling_on_sc=False),
)
def _embed_kernel(x_hbm, tok_hbm, pos_hbm, out_hbm,
                  idx_v, buf0, buf1, pos_v, gsem0, gsem1, osem0, osem1):
    wid = lax.axis_index("s") * NC + lax.axis_index("c")
    batch_base = wid * BW
    bufs = (buf0, buf1)
    gsems = (gsem0, gsem1)
    osems = (osem0, osem1)

    pltpu.sync_copy(pos_hbm, pos_v)

    def gather_parts(c, slot):
        brow = batch_base + c * BPC
        parts = []
        for b in range(BPC):
            for off, size in SUBS:
                parts.append((
                    tok_hbm.at[idx_v.at[slot * BPC + b, pl.ds(off, size)]],
                    bufs[slot].at[b, pl.ds(off, size)],
                    gsems[slot],
                ))
        return brow, parts

    def issue_gather(c, slot):
        brow, parts = gather_parts(c, slot)
        pltpu.sync_copy(x_hbm.at[pl.ds(brow, BPC)],
                        idx_v.at[pl.ds(slot * BPC, BPC)])
        for src, dst, sem in parts:
            pltpu.async_copy(src, dst, sem)

    def wait_gather(c, slot):
        _, parts = gather_parts(c, slot)
        for src, dst, sem in parts:
            pltpu.make_async_copy(src, dst, sem).wait()

    def issue_out(c, slot):
        brow = batch_base + c * BPC
        pltpu.async_copy(bufs[slot], out_hbm.at[pl.ds(brow, BPC)], osems[slot])

    def wait_out(slot):
        pltpu.make_async_copy(bufs[slot], out_hbm.at[pl.ds(batch_base, BPC)],
                              osems[slot]).wait()

    issue_gather(0, 0)

    def pair_body(i, carry):
        c0 = 2 * i
        for slot in range(2):
            c = c0 + slot
            nslot = 1 - slot
            nxt = c + 1

            @pl.when(nxt < NCHUNK)
            def _prefetch():
                @pl.when(c >= 1)
                def _reclaim():
                    wait_out(nslot)
                issue_gather(nxt, nslot)

            wait_gather(c, slot)
            buf = bufs[slot]

            def pos_add(p, carry2):
                for q in range(EMBED // 16):
                    sl = pl.ds(q * 16, 16)
                    pv = pos_v[p, sl]
                    for b in range(BPC):
                        buf[b, p, sl] = buf[b, p, sl] + pv
                return carry2

            lax.fori_loop(0, MAXLEN, pos_add, 0)
            issue_out(c, slot)
        return carry

    lax.fori_loop(0, NCHUNK // 2, pair_body, 0)
    wait_out(0)
    wait_out(1)


def kernel(x, token_table, pos_table):
    return _embed_kernel(x.astype(jnp.int32), token_table, pos_table)
